# baseline (speedup 1.0000x reference)
import functools

import jax
import jax.numpy as jnp
from jax import lax
from jax.experimental import pallas as pl
from jax.experimental.pallas import tpu as pltpu
from jax.experimental.pallas import tpu_sc as plsc

VOCAB = 100000
D = 768
SEQ = 200
BATCH = 1024

NC = 2
NS = 16
NW = NC * NS
BPW = BATCH // NW
PBLK = 40
NP = SEQ // PBLK
LANES = 16
NBUF = 3
MAIN = (BPW // NBUF) * NBUF


def _pos_encoding(max_seq_len, d_model):
    even_i = jnp.arange(0, d_model, 2, dtype=jnp.float32)
    denominator = jnp.power(10000.0, even_i / d_model)
    position = jnp.arange(max_seq_len, dtype=jnp.float32).reshape(max_seq_len, 1)
    even_pe = jnp.sin(position / denominator)
    odd_pe = jnp.cos(position / denominator)
    stacked = jnp.stack([even_pe, odd_pe], axis=2)
    return stacked.reshape(max_seq_len, d_model)


def _sc_body(x_hbm, pe_hbm, table_hbm, out_hbm, idx_v, pe_v,
             rows0, rows1, rows2, gsem0, gsem1, gsem2, osem0, osem1, osem2):
    wid = lax.axis_index("s") * NC + lax.axis_index("c")
    b0_w = wid * BPW
    rows = (rows0, rows1, rows2)
    gsems = (gsem0, gsem1, gsem2)
    osems = (osem0, osem1, osem2)

    def start_gather(c, p, j):
        off = c * SEQ + p * PBLK
        pltpu.async_copy(table_hbm.at[idx_v.at[pl.ds(off, PBLK)]],
                         rows[j], gsems[j])

    def wait_gather(j):
        pltpu.make_async_copy(pe_hbm.at[pl.ds(0, PBLK)], rows[j], gsems[j]).wait()

    def start_scatter(c, p, j):
        out_off = (b0_w + c) * SEQ + p * PBLK
        pltpu.async_copy(rows[j], out_hbm.at[pl.ds(out_off, PBLK)], osems[j])

    def wait_scatter(j):
        pltpu.make_async_copy(rows[j], out_hbm.at[pl.ds(0, PBLK)], osems[j]).wait()

    def add_pe(j):
        r = rows[j]

        @plsc.parallel_loop(0, PBLK, unroll=2)
        def _(i):
            for k in range(D // LANES):
                sl = pl.ds(k * LANES, LANES)
                plsc.addupdate(r.at[i, sl], pe_v[i, sl])

    pltpu.sync_copy(x_hbm.at[pl.ds(b0_w * SEQ, BPW * SEQ)], idx_v)

    def pass_body(p, _):
        pltpu.sync_copy(pe_hbm.at[pl.ds(p * PBLK, PBLK)], pe_v)
        for j in range(NBUF):
            start_gather(j, p, j)

        def body(i, _):
            for j in range(NBUF):
                c = NBUF * i + j
                wait_gather(j)
                add_pe(j)
                start_scatter(c, p, j)

                @pl.when(c + NBUF < BPW)
                def _():
                    wait_scatter(j)
                    start_gather(c + NBUF, p, j)

            return 0

        lax.fori_loop(0, MAIN // NBUF, body, 0)
        for c in range(MAIN, BPW):
            j = c - MAIN
            wait_gather(j)
            add_pe(j)
            start_scatter(c, p, j)
        for j in range(NBUF):
            wait_scatter(j)
        return 0

    lax.fori_loop(0, NP, pass_body, 0)


@jax.jit
def _sc_call(x2d, pe, table):
    mesh = plsc.VectorSubcoreMesh(core_axis_name="c", subcore_axis_name="s")
    return pl.kernel(
        _sc_body,
        out_type=jax.ShapeDtypeStruct((BATCH * SEQ, D), jnp.float32),
        mesh=mesh,
        scratch_types=[
            pltpu.VMEM((BPW * SEQ,), jnp.int32),
            pltpu.VMEM((PBLK, D), jnp.float32),
            pltpu.VMEM((PBLK, D), jnp.float32),
            pltpu.VMEM((PBLK, D), jnp.float32),
            pltpu.VMEM((PBLK, D), jnp.float32),
            pltpu.SemaphoreType.DMA,
            pltpu.SemaphoreType.DMA,
            pltpu.SemaphoreType.DMA,
            pltpu.SemaphoreType.DMA,
            pltpu.SemaphoreType.DMA,
            pltpu.SemaphoreType.DMA,
        ],
    )(x2d, pe, table)


def kernel(x, table):
    pe = _pos_encoding(SEQ, D)
    x_flat = x.reshape(-1).astype(jnp.int32)
    out = _sc_call(x_flat, pe, table)
    return out.reshape(BATCH, SEQ, D)


# speedup vs baseline: 1.0104x; 1.0104x over previous
import functools

import jax
import jax.numpy as jnp
from jax import lax
from jax.experimental import pallas as pl
from jax.experimental.pallas import tpu as pltpu
from jax.experimental.pallas import tpu_sc as plsc

VOCAB = 100000
D = 768
SEQ = 200
BATCH = 1024

NC = 2
NS = 16
NW = NC * NS
BPW = BATCH // NW
PBLK = 40
NP = SEQ // PBLK
LANES = 16


def _pos_encoding(max_seq_len, d_model):
    even_i = jnp.arange(0, d_model, 2, dtype=jnp.float32)
    denominator = jnp.power(10000.0, even_i / d_model)
    position = jnp.arange(max_seq_len, dtype=jnp.float32).reshape(max_seq_len, 1)
    even_pe = jnp.sin(position / denominator)
    odd_pe = jnp.cos(position / denominator)
    stacked = jnp.stack([even_pe, odd_pe], axis=2)
    return stacked.reshape(max_seq_len, d_model)


def _sc_body(x_hbm, pe_hbm, table_hbm, out_hbm, idx_v, pe_v, rows0, rows1,
             gsem0, gsem1, osem0, osem1):
    wid = lax.axis_index("s") * NC + lax.axis_index("c")
    row0_w = wid * (BPW * SEQ)
    pltpu.sync_copy(x_hbm.at[pl.ds(row0_w, BPW * SEQ)], idx_v)

    def start_gather(off, rows, sem):
        pltpu.async_copy(table_hbm.at[idx_v.at[pl.ds(off, PBLK)]], rows, sem)

    def wait_gather(rows, sem):
        pltpu.make_async_copy(pe_hbm.at[pl.ds(0, PBLK)], rows, sem).wait()

    def start_scatter(out_off, rows, sem):
        pltpu.async_copy(rows, out_hbm.at[pl.ds(out_off, PBLK)], sem)

    def wait_scatter(rows, sem):
        pltpu.make_async_copy(rows, out_hbm.at[pl.ds(0, PBLK)], sem).wait()

    def add_pe(rows):
        def add_row(j, _):
            for k in range(D // LANES):
                sl = pl.ds(k * LANES, LANES)
                plsc.addupdate(rows.at[j, sl], pe_v[j, sl])
            return 0

        lax.fori_loop(0, PBLK, add_row, 0)

    for p in range(NP):
        pltpu.sync_copy(pe_hbm.at[pl.ds(p * PBLK, PBLK)], pe_v)
        start_gather(0 * SEQ + p * PBLK, rows0, gsem0)

        def body(bb, _):
            b0 = 2 * bb
            b1 = 2 * bb + 1
            wait_gather(rows0, gsem0)

            @pl.when(bb > 0)
            def _():
                wait_scatter(rows1, osem1)

            start_gather(b1 * SEQ + p * PBLK, rows1, gsem1)
            add_pe(rows0)
            start_scatter(row0_w + b0 * SEQ + p * PBLK, rows0, osem0)
            wait_gather(rows1, gsem1)

            @pl.when(bb < BPW // 2 - 1)
            def _():
                wait_scatter(rows0, osem0)
                start_gather((b0 + 2) * SEQ + p * PBLK, rows0, gsem0)

            add_pe(rows1)
            start_scatter(row0_w + b1 * SEQ + p * PBLK, rows1, osem1)
            return 0

        lax.fori_loop(0, BPW // 2, body, 0)
        wait_scatter(rows0, osem0)
        wait_scatter(rows1, osem1)


@jax.jit
def _sc_call(x_flat, pe, table):
    mesh = plsc.VectorSubcoreMesh(core_axis_name="c", subcore_axis_name="s")
    return pl.kernel(
        _sc_body,
        out_type=jax.ShapeDtypeStruct((BATCH * SEQ, D), jnp.float32),
        mesh=mesh,
        scratch_types=[
            pltpu.VMEM((BPW * SEQ,), jnp.int32),
            pltpu.VMEM((PBLK, D), jnp.float32),
            pltpu.VMEM((PBLK, D), jnp.float32),
            pltpu.VMEM((PBLK, D), jnp.float32),
            pltpu.SemaphoreType.DMA,
            pltpu.SemaphoreType.DMA,
            pltpu.SemaphoreType.DMA,
            pltpu.SemaphoreType.DMA,
        ],
    )(x_flat, pe, table)


def kernel(x, table):
    pe = _pos_encoding(SEQ, D)
    x_flat = x.reshape(-1).astype(jnp.int32)
    out = _sc_call(x_flat, pe, table)
    return out.reshape(BATCH, SEQ, D)


# speedup vs baseline: 1.1962x; 1.1839x over previous
import functools

import jax
import jax.numpy as jnp
from jax import lax
from jax.experimental import pallas as pl
from jax.experimental.pallas import tpu as pltpu
from jax.experimental.pallas import tpu_sc as plsc

VOCAB = 100000
D = 768
SEQ = 200
BATCH = 1024

NC = 2
NS = 16
NW = NC * NS
BPW = BATCH // NW
PBLK = 40
NP = SEQ // PBLK
LANES = 16


def _pos_encoding(max_seq_len, d_model):
    even_i = jnp.arange(0, d_model, 2, dtype=jnp.float32)
    denominator = jnp.power(10000.0, even_i / d_model)
    position = jnp.arange(max_seq_len, dtype=jnp.float32).reshape(max_seq_len, 1)
    even_pe = jnp.sin(position / denominator)
    odd_pe = jnp.cos(position / denominator)
    stacked = jnp.stack([even_pe, odd_pe], axis=2)
    return stacked.reshape(max_seq_len, d_model)


def _sc_body(x_hbm, pe_hbm, table_hbm, out_hbm, idx_v, pe_v, rows0, rows1,
             gsem0, gsem1, osem0, osem1):
    wid = lax.axis_index("s") * NC + lax.axis_index("c")
    row0_w = wid * (BPW * SEQ)
    pltpu.sync_copy(x_hbm.at[pl.ds(row0_w, BPW * SEQ)], idx_v)

    def start_gather(off, rows, sem):
        pltpu.async_copy(table_hbm.at[idx_v.at[pl.ds(off, PBLK)]], rows, sem)

    def wait_gather(rows, sem):
        pltpu.make_async_copy(pe_hbm.at[pl.ds(0, PBLK)], rows, sem).wait()

    def start_scatter(out_off, rows, sem):
        pltpu.async_copy(rows, out_hbm.at[pl.ds(out_off, PBLK)], sem)

    def wait_scatter(rows, sem):
        pltpu.make_async_copy(rows, out_hbm.at[pl.ds(0, PBLK)], sem).wait()

    def add_pe_part(rows, lo, hi):
        def add_row(j, _):
            for k in range(D // LANES):
                sl = pl.ds(k * LANES, LANES)
                plsc.addupdate(rows.at[j, sl], pe_v[j, sl])
            return 0

        lax.fori_loop(lo, hi, add_row, 0)

    def add_scatter(rows, out_off, sem):
        for lo, hi in ((0, 24), (24, PBLK)):
            add_pe_part(rows, lo, hi)
            pltpu.async_copy(
                rows.at[pl.ds(lo, hi - lo)],
                out_hbm.at[pl.ds(out_off + lo, hi - lo)], sem)

    for p in range(NP):
        pltpu.sync_copy(pe_hbm.at[pl.ds(p * PBLK, PBLK)], pe_v)
        start_gather(0 * SEQ + p * PBLK, rows0, gsem0)

        def body(bb, _):
            b0 = 2 * bb
            b1 = 2 * bb + 1
            wait_gather(rows0, gsem0)

            @pl.when(bb > 0)
            def _():
                wait_scatter(rows1, osem1)

            start_gather(b1 * SEQ + p * PBLK, rows1, gsem1)
            add_scatter(rows0, row0_w + b0 * SEQ + p * PBLK, osem0)
            wait_gather(rows1, gsem1)

            @pl.when(bb < BPW // 2 - 1)
            def _():
                wait_scatter(rows0, osem0)
                start_gather((b0 + 2) * SEQ + p * PBLK, rows0, gsem0)

            add_scatter(rows1, row0_w + b1 * SEQ + p * PBLK, osem1)
            return 0

        lax.fori_loop(0, BPW // 2, body, 0)
        wait_scatter(rows0, osem0)
        wait_scatter(rows1, osem1)


@jax.jit
def _sc_call(x_flat, pe, table):
    mesh = plsc.VectorSubcoreMesh(core_axis_name="c", subcore_axis_name="s")
    return pl.kernel(
        _sc_body,
        out_type=jax.ShapeDtypeStruct((BATCH * SEQ, D), jnp.float32),
        mesh=mesh,
        scratch_types=[
            pltpu.VMEM((BPW * SEQ,), jnp.int32),
            pltpu.VMEM((PBLK, D), jnp.float32),
            pltpu.VMEM((PBLK, D), jnp.float32),
            pltpu.VMEM((PBLK, D), jnp.float32),
            pltpu.SemaphoreType.DMA,
            pltpu.SemaphoreType.DMA,
            pltpu.SemaphoreType.DMA,
            pltpu.SemaphoreType.DMA,
        ],
    )(x_flat, pe, table)


def kernel(x, table):
    pe = _pos_encoding(SEQ, D)
    x_flat = x.reshape(-1).astype(jnp.int32)
    out = _sc_call(x_flat, pe, table)
    return out.reshape(BATCH, SEQ, D)
